# baseline (speedup 1.0000x reference)
import functools

import jax
import jax.numpy as jnp
from jax import lax
from jax.experimental import pallas as pl
from jax.experimental.pallas import tpu as pltpu
from jax.experimental.pallas import tpu_sc as plsc

NUM_NODES = 50000
NUM_EDGES = 1600000
N_ROWS = 392
N_PAD = N_ROWS * 128
NW = 32
CHUNK = 3200
N_CHUNKS = NUM_EDGES // CHUNK
FULL_ROUNDS = N_CHUNKS // NW
EXTRA = N_CHUNKS - FULL_ROUNDS * NW

_mesh = plsc.VectorSubcoreMesh(core_axis_name="c", subcore_axis_name="s")
_sc_params = pltpu.CompilerParams(needs_layout_passes=False)


def _wid():
    return lax.axis_index("s") * 2 + lax.axis_index("c")


@functools.partial(
    pl.kernel,
    mesh=_mesh,
    out_type=jax.ShapeDtypeStruct((NW * N_PAD,), jnp.float32),
    compiler_params=_sc_params,
    scratch_types=[
        pltpu.VMEM((2, CHUNK), jnp.int32),
        pltpu.VMEM((N_PAD,), jnp.float32),
        pltpu.SemaphoreType.DMA,
    ],
)
def _degree_kernel(ei_hbm, deg_out_hbm, ei_v, deg_v, sem):
    wid = _wid()

    zeros = jnp.zeros((16,), jnp.float32)

    @plsc.parallel_loop(0, N_PAD, 16, unroll=4)
    def _zero(i):
        deg_v[pl.ds(i, 16)] = zeros

    ones = jnp.ones((16,), jnp.float32)

    def do_chunk(k):
        pltpu.async_copy(ei_hbm.at[:, pl.ds(k * CHUNK, CHUNK)], ei_v, sem).wait()

        @plsc.parallel_loop(0, CHUNK, 16, unroll=8)
        def _accum(i):
            idx = ei_v[1, pl.ds(i, 16)]
            plsc.addupdate_scatter(deg_v, [idx], ones)

    for r in range(FULL_ROUNDS):
        do_chunk(r * NW + wid)

    @pl.when(wid < EXTRA)
    def _tail():
        do_chunk(FULL_ROUNDS * NW + wid)

    pltpu.sync_copy(deg_v, deg_out_hbm.at[pl.ds(wid * N_PAD, N_PAD)])


def _reduce_rsqrt_body(p_ref, o_ref):
    s = jnp.sum(p_ref[...], axis=0)
    o_ref[...] = jnp.where(s > 0.0, jax.lax.rsqrt(s), 0.0)


@functools.partial(
    pl.kernel,
    mesh=_mesh,
    out_type=jax.ShapeDtypeStruct((NUM_EDGES,), jnp.float32),
    compiler_params=_sc_params,
    scratch_types=[
        pltpu.VMEM((N_PAD,), jnp.float32),
        pltpu.VMEM((2, CHUNK), jnp.int32),
        pltpu.VMEM((CHUNK,), jnp.float32),
        pltpu.SemaphoreType.DMA,
        pltpu.SemaphoreType.DMA,
    ],
)
def _norm_kernel(ei_hbm, tab_hbm, out_hbm, tab_v, ei_v, out_v, sem_tab, sem):
    wid = _wid()
    pltpu.sync_copy(tab_hbm, tab_v)

    def do_chunk(k):
        pltpu.async_copy(ei_hbm.at[:, pl.ds(k * CHUNK, CHUNK)], ei_v, sem).wait()

        @plsc.parallel_loop(0, CHUNK, 16, unroll=8)
        def _gather(i):
            ri = ei_v[0, pl.ds(i, 16)]
            ci = ei_v[1, pl.ds(i, 16)]
            r = plsc.load_gather(tab_v, [ri])
            c = plsc.load_gather(tab_v, [ci])
            out_v[pl.ds(i, 16)] = r * c

        pltpu.sync_copy(out_v, out_hbm.at[pl.ds(k * CHUNK, CHUNK)])

    for r in range(FULL_ROUNDS):
        do_chunk(r * NW + wid)

    @pl.when(wid < EXTRA)
    def _tail():
        do_chunk(FULL_ROUNDS * NW + wid)


def kernel(edge_index):
    ei = edge_index.astype(jnp.int32)
    partials = _degree_kernel(ei)
    deg_inv = pl.pallas_call(
        _reduce_rsqrt_body,
        out_shape=jax.ShapeDtypeStruct((N_ROWS, 128), jnp.float32),
    )(partials.reshape(NW, N_ROWS, 128))
    return _norm_kernel(ei, deg_inv.reshape(N_PAD))


# speedup vs baseline: 1.2532x; 1.2532x over previous
import functools

import jax
import jax.numpy as jnp
from jax import lax
from jax.experimental import pallas as pl
from jax.experimental.pallas import tpu as pltpu
from jax.experimental.pallas import tpu_sc as plsc

NUM_NODES = 50000
NUM_EDGES = 1600000
N_ROWS = 392
N_PAD = N_ROWS * 128
NW = 32
CHUNK = 3200
N_CHUNKS = NUM_EDGES // CHUNK
FULL_ROUNDS = N_CHUNKS // NW
EXTRA = N_CHUNKS - FULL_ROUNDS * NW
ROUNDS = FULL_ROUNDS + 1

_mesh = plsc.VectorSubcoreMesh(core_axis_name="c", subcore_axis_name="s")
_sc_params = pltpu.CompilerParams(needs_layout_passes=False)


def _wid():
    return lax.axis_index("s") * 2 + lax.axis_index("c")


def _chunk_of(r, wid):
    return jnp.minimum(r * NW + wid, N_CHUNKS - 1)


@functools.partial(
    pl.kernel,
    mesh=_mesh,
    out_type=jax.ShapeDtypeStruct((NW * N_PAD,), jnp.float32),
    compiler_params=_sc_params,
    scratch_types=[
        pltpu.VMEM((2, CHUNK), jnp.int32),
        pltpu.VMEM((2, CHUNK), jnp.int32),
        pltpu.VMEM((N_PAD,), jnp.float32),
        pltpu.SemaphoreType.DMA,
        pltpu.SemaphoreType.DMA,
    ],
)
def _degree_kernel(ei_hbm, deg_out_hbm, ei_a, ei_b, deg_v, sem0, sem1):
    wid = _wid()
    sems = (sem0, sem1)
    bufs = (ei_a, ei_b)

    def start_in(r):
        k = _chunk_of(r, wid)
        buf = r % 2
        return pltpu.async_copy(
            ei_hbm.at[:, pl.ds(k * CHUNK, CHUNK)], bufs[buf], sems[buf]
        )

    pending = start_in(0)

    zeros = jnp.zeros((16,), jnp.float32)

    @plsc.parallel_loop(0, N_PAD, 16, unroll=4)
    def _zero(i):
        deg_v[pl.ds(i, 16)] = zeros

    ones = jnp.ones((16,), jnp.float32)

    for r in range(ROUNDS):
        buf = r % 2
        pending.wait()
        if r + 1 < ROUNDS:
            pending = start_in(r + 1)

        cbuf = bufs[buf]

        def accum_chunk(cbuf=cbuf):
            @plsc.parallel_loop(0, CHUNK, 16, unroll=8)
            def _accum(i):
                idx = cbuf[1, pl.ds(i, 16)]
                plsc.addupdate_scatter(deg_v, [idx], ones)

        if r < FULL_ROUNDS:
            accum_chunk()
        else:
            pl.when(wid < EXTRA)(accum_chunk)

    pltpu.sync_copy(deg_v, deg_out_hbm.at[pl.ds(wid * N_PAD, N_PAD)])


def _reduce_rsqrt_body(p_ref, o_ref):
    s = jnp.sum(p_ref[...], axis=0)
    o_ref[...] = jnp.where(s > 0.0, jax.lax.rsqrt(s), 0.0)


@functools.partial(
    pl.kernel,
    mesh=_mesh,
    out_type=jax.ShapeDtypeStruct((NUM_EDGES,), jnp.float32),
    compiler_params=_sc_params,
    scratch_types=[
        pltpu.VMEM((N_PAD,), jnp.float32),
        pltpu.VMEM((2, CHUNK), jnp.int32),
        pltpu.VMEM((2, CHUNK), jnp.int32),
        pltpu.VMEM((CHUNK,), jnp.float32),
        pltpu.VMEM((CHUNK,), jnp.float32),
        pltpu.SemaphoreType.DMA,
        pltpu.SemaphoreType.DMA,
        pltpu.SemaphoreType.DMA,
        pltpu.SemaphoreType.DMA,
        pltpu.SemaphoreType.DMA,
    ],
)
def _norm_kernel(
    ei_hbm, tab_hbm, out_hbm, tab_v, ei_a, ei_b, out_a, out_b,
    sem_tab, si0, si1, so0, so1
):
    wid = _wid()
    sems_in = (si0, si1)
    sems_out = (so0, so1)
    in_bufs = (ei_a, ei_b)
    out_bufs = (out_a, out_b)

    def start_in(r):
        k = _chunk_of(r, wid)
        buf = r % 2
        return pltpu.async_copy(
            ei_hbm.at[:, pl.ds(k * CHUNK, CHUNK)], in_bufs[buf], sems_in[buf]
        )

    pending = start_in(0)
    tab_cp = pltpu.async_copy(tab_hbm, tab_v, sem_tab)
    tab_cp.wait()

    out_pending = [None, None]
    for r in range(ROUNDS):
        buf = r % 2
        pending.wait()
        if r + 1 < ROUNDS:
            pending = start_in(r + 1)
        if out_pending[buf] is not None:
            out_pending[buf].wait()
            out_pending[buf] = None

        cbuf = in_bufs[buf]
        obuf = out_bufs[buf]
        k = _chunk_of(r, wid)

        def gather_chunk(cbuf=cbuf, obuf=obuf):
            @plsc.parallel_loop(0, CHUNK, 16, unroll=8)
            def _gather(i):
                ri = cbuf[0, pl.ds(i, 16)]
                ci = cbuf[1, pl.ds(i, 16)]
                r_ = plsc.load_gather(tab_v, [ri])
                c_ = plsc.load_gather(tab_v, [ci])
                obuf[pl.ds(i, 16)] = r_ * c_

        if r < FULL_ROUNDS:
            gather_chunk()
            out_pending[buf] = pltpu.async_copy(
                obuf, out_hbm.at[pl.ds(k * CHUNK, CHUNK)], sems_out[buf]
            )
        else:

            def tail(cbuf=cbuf, obuf=obuf, k=k):
                gather_chunk(cbuf, obuf)
                pltpu.sync_copy(obuf, out_hbm.at[pl.ds(k * CHUNK, CHUNK)])

            pl.when(wid < EXTRA)(tail)

    for cp in out_pending:
        if cp is not None:
            cp.wait()


def kernel(edge_index):
    ei = edge_index.astype(jnp.int32)
    partials = _degree_kernel(ei)
    deg_inv = pl.pallas_call(
        _reduce_rsqrt_body,
        out_shape=jax.ShapeDtypeStruct((N_ROWS, 128), jnp.float32),
    )(partials.reshape(NW, N_ROWS, 128))
    return _norm_kernel(ei, deg_inv.reshape(N_PAD))


# speedup vs baseline: 1.5035x; 1.1997x over previous
import functools

import jax
import jax.numpy as jnp
from jax import lax
from jax.experimental import pallas as pl
from jax.experimental.pallas import tpu as pltpu
from jax.experimental.pallas import tpu_sc as plsc

NUM_NODES = 50000
NUM_EDGES = 1600000
N_ROWS = 392
N_PAD = N_ROWS * 128
NW = 32
CHUNK = 6400
N_CHUNKS = NUM_EDGES // CHUNK
FULL_ROUNDS = N_CHUNKS // NW
EXTRA = N_CHUNKS - FULL_ROUNDS * NW
ROUNDS = FULL_ROUNDS + 1

_mesh = plsc.VectorSubcoreMesh(core_axis_name="c", subcore_axis_name="s")
_sc_params = pltpu.CompilerParams(needs_layout_passes=False)


def _wid():
    return lax.axis_index("s") * 2 + lax.axis_index("c")


def _chunk_of(r, wid):
    return jnp.minimum(r * NW + wid, N_CHUNKS - 1)


@functools.partial(
    pl.kernel,
    mesh=_mesh,
    out_type=jax.ShapeDtypeStruct((NW * N_PAD,), jnp.float32),
    compiler_params=_sc_params,
    scratch_types=[
        pltpu.VMEM((2, CHUNK), jnp.int32),
        pltpu.VMEM((2, CHUNK), jnp.int32),
        pltpu.VMEM((N_PAD,), jnp.float32),
        pltpu.SemaphoreType.DMA,
        pltpu.SemaphoreType.DMA,
    ],
)
def _degree_kernel(ei_hbm, deg_out_hbm, ei_a, ei_b, deg_v, sem0, sem1):
    wid = _wid()
    sems = (sem0, sem1)
    bufs = (ei_a, ei_b)

    def start_in(r):
        k = _chunk_of(r, wid)
        buf = r % 2
        return pltpu.async_copy(
            ei_hbm.at[:, pl.ds(k * CHUNK, CHUNK)], bufs[buf], sems[buf]
        )

    pending = start_in(0)

    zeros = jnp.zeros((16,), jnp.float32)

    @plsc.parallel_loop(0, N_PAD, 16, unroll=4)
    def _zero(i):
        deg_v[pl.ds(i, 16)] = zeros

    ones = jnp.ones((16,), jnp.float32)

    for r in range(ROUNDS):
        buf = r % 2
        pending.wait()
        if r + 1 < ROUNDS:
            pending = start_in(r + 1)

        cbuf = bufs[buf]

        def accum_chunk(cbuf=cbuf):
            @plsc.parallel_loop(0, CHUNK, 16, unroll=8)
            def _accum(i):
                idx = cbuf[1, pl.ds(i, 16)]
                plsc.addupdate_scatter(deg_v, [idx], ones)

        if r < FULL_ROUNDS:
            accum_chunk()
        else:
            pl.when(wid < EXTRA)(accum_chunk)

    pltpu.sync_copy(deg_v, deg_out_hbm.at[pl.ds(wid * N_PAD, N_PAD)])


def _reduce_rsqrt_body(p_ref, o_ref):
    s = jnp.sum(p_ref[...], axis=0)
    o_ref[...] = jnp.where(s > 0.0, jax.lax.rsqrt(s), 0.0)


@functools.partial(
    pl.kernel,
    mesh=_mesh,
    out_type=jax.ShapeDtypeStruct((NUM_EDGES,), jnp.float32),
    compiler_params=_sc_params,
    scratch_types=[
        pltpu.VMEM((N_PAD,), jnp.float32),
        pltpu.VMEM((2, CHUNK), jnp.int32),
        pltpu.VMEM((2, CHUNK), jnp.int32),
        pltpu.VMEM((CHUNK,), jnp.float32),
        pltpu.VMEM((CHUNK,), jnp.float32),
        pltpu.SemaphoreType.DMA,
        pltpu.SemaphoreType.DMA,
        pltpu.SemaphoreType.DMA,
        pltpu.SemaphoreType.DMA,
        pltpu.SemaphoreType.DMA,
    ],
)
def _norm_kernel(
    ei_hbm, tab_hbm, out_hbm, tab_v, ei_a, ei_b, out_a, out_b,
    sem_tab, si0, si1, so0, so1
):
    wid = _wid()
    sems_in = (si0, si1)
    sems_out = (so0, so1)
    in_bufs = (ei_a, ei_b)
    out_bufs = (out_a, out_b)

    def start_in(r):
        k = _chunk_of(r, wid)
        buf = r % 2
        return pltpu.async_copy(
            ei_hbm.at[:, pl.ds(k * CHUNK, CHUNK)], in_bufs[buf], sems_in[buf]
        )

    pending = start_in(0)
    tab_cp = pltpu.async_copy(tab_hbm, tab_v, sem_tab)
    tab_cp.wait()

    out_pending = [None, None]
    for r in range(ROUNDS):
        buf = r % 2
        pending.wait()
        if r + 1 < ROUNDS:
            pending = start_in(r + 1)
        if out_pending[buf] is not None:
            out_pending[buf].wait()
            out_pending[buf] = None

        cbuf = in_bufs[buf]
        obuf = out_bufs[buf]
        k = _chunk_of(r, wid)

        def gather_chunk(cbuf=cbuf, obuf=obuf):
            @plsc.parallel_loop(0, CHUNK, 16, unroll=8)
            def _gather(i):
                ri = cbuf[0, pl.ds(i, 16)]
                ci = cbuf[1, pl.ds(i, 16)]
                r_ = plsc.load_gather(tab_v, [ri])
                c_ = plsc.load_gather(tab_v, [ci])
                obuf[pl.ds(i, 16)] = r_ * c_

        if r < FULL_ROUNDS:
            gather_chunk()
            out_pending[buf] = pltpu.async_copy(
                obuf, out_hbm.at[pl.ds(k * CHUNK, CHUNK)], sems_out[buf]
            )
        else:

            def tail(cbuf=cbuf, obuf=obuf, k=k):
                gather_chunk(cbuf, obuf)
                pltpu.sync_copy(obuf, out_hbm.at[pl.ds(k * CHUNK, CHUNK)])

            pl.when(wid < EXTRA)(tail)

    for cp in out_pending:
        if cp is not None:
            cp.wait()


def kernel(edge_index):
    ei = edge_index.astype(jnp.int32)
    partials = _degree_kernel(ei)
    deg_inv = pl.pallas_call(
        _reduce_rsqrt_body,
        out_shape=jax.ShapeDtypeStruct((N_ROWS, 128), jnp.float32),
    )(partials.reshape(NW, N_ROWS, 128))
    return _norm_kernel(ei, deg_inv.reshape(N_PAD))


# speedup vs baseline: 1.5602x; 1.0377x over previous
import functools

import jax
import jax.numpy as jnp
from jax import lax
from jax.experimental import pallas as pl
from jax.experimental.pallas import tpu as pltpu
from jax.experimental.pallas import tpu_sc as plsc

NUM_NODES = 50000
NUM_EDGES = 1600000
N_ROWS = 392
N_PAD = N_ROWS * 128
NW = 32
def _sched(chunk):
    n_chunks = NUM_EDGES // chunk
    full = n_chunks // NW
    extra = n_chunks - full * NW
    return n_chunks, full, extra

D_CHUNK = 12800
D_N_CHUNKS, D_FULL, D_EXTRA = _sched(D_CHUNK)
CHUNK = 6400
N_CHUNKS, FULL_ROUNDS, EXTRA = _sched(CHUNK)
ROUNDS = FULL_ROUNDS + 1

_mesh = plsc.VectorSubcoreMesh(core_axis_name="c", subcore_axis_name="s")
_sc_params = pltpu.CompilerParams(needs_layout_passes=False)


def _wid():
    return lax.axis_index("s") * 2 + lax.axis_index("c")


def _chunk_of(r, wid, n_chunks=None):
    return jnp.minimum(r * NW + wid, (n_chunks or N_CHUNKS) - 1)


@functools.partial(
    pl.kernel,
    mesh=_mesh,
    out_type=jax.ShapeDtypeStruct((NW * N_PAD,), jnp.float32),
    compiler_params=_sc_params,
    scratch_types=[
        pltpu.VMEM((2, D_CHUNK), jnp.int32),
        pltpu.VMEM((2, D_CHUNK), jnp.int32),
        pltpu.VMEM((N_PAD,), jnp.float32),
        pltpu.SemaphoreType.DMA,
        pltpu.SemaphoreType.DMA,
    ],
)
def _degree_kernel(ei_hbm, deg_out_hbm, ei_a, ei_b, deg_v, sem0, sem1):
    wid = _wid()
    sems = (sem0, sem1)
    bufs = (ei_a, ei_b)

    def start_in(r):
        k = _chunk_of(r, wid, D_N_CHUNKS)
        buf = r % 2
        return pltpu.async_copy(
            ei_hbm.at[:, pl.ds(k * D_CHUNK, D_CHUNK)], bufs[buf], sems[buf]
        )

    pending = start_in(0)

    zeros = jnp.zeros((16,), jnp.float32)

    @plsc.parallel_loop(0, N_PAD, 16, unroll=4)
    def _zero(i):
        deg_v[pl.ds(i, 16)] = zeros

    ones = jnp.ones((16,), jnp.float32)

    for r in range(D_FULL + 1):
        buf = r % 2
        pending.wait()
        if r + 1 < D_FULL + 1:
            pending = start_in(r + 1)

        cbuf = bufs[buf]

        def accum_chunk(cbuf=cbuf):
            @plsc.parallel_loop(0, D_CHUNK, 16, unroll=8)
            def _accum(i):
                idx = cbuf[1, pl.ds(i, 16)]
                plsc.addupdate_scatter(deg_v, [idx], ones)

        if r < D_FULL:
            accum_chunk()
        else:
            pl.when(wid < D_EXTRA)(accum_chunk)

    pltpu.sync_copy(deg_v, deg_out_hbm.at[pl.ds(wid * N_PAD, N_PAD)])


def _reduce_rsqrt_body(p_ref, o_ref):
    s = jnp.sum(p_ref[...], axis=0)
    o_ref[...] = jnp.where(s > 0.0, jax.lax.rsqrt(s), 0.0)


@functools.partial(
    pl.kernel,
    mesh=_mesh,
    out_type=jax.ShapeDtypeStruct((NUM_EDGES,), jnp.float32),
    compiler_params=_sc_params,
    scratch_types=[
        pltpu.VMEM((N_PAD,), jnp.float32),
        pltpu.VMEM((2, CHUNK), jnp.int32),
        pltpu.VMEM((2, CHUNK), jnp.int32),
        pltpu.VMEM((CHUNK,), jnp.float32),
        pltpu.VMEM((CHUNK,), jnp.float32),
        pltpu.SemaphoreType.DMA,
        pltpu.SemaphoreType.DMA,
        pltpu.SemaphoreType.DMA,
        pltpu.SemaphoreType.DMA,
        pltpu.SemaphoreType.DMA,
    ],
)
def _norm_kernel(
    ei_hbm, tab_hbm, out_hbm, tab_v, ei_a, ei_b, out_a, out_b,
    sem_tab, si0, si1, so0, so1
):
    wid = _wid()
    sems_in = (si0, si1)
    sems_out = (so0, so1)
    in_bufs = (ei_a, ei_b)
    out_bufs = (out_a, out_b)

    def start_in(r):
        k = _chunk_of(r, wid)
        buf = r % 2
        return pltpu.async_copy(
            ei_hbm.at[:, pl.ds(k * CHUNK, CHUNK)], in_bufs[buf], sems_in[buf]
        )

    pending = start_in(0)
    tab_cp = pltpu.async_copy(tab_hbm, tab_v, sem_tab)
    tab_cp.wait()

    out_pending = [None, None]
    for r in range(ROUNDS):
        buf = r % 2
        pending.wait()
        if r + 1 < ROUNDS:
            pending = start_in(r + 1)
        if out_pending[buf] is not None:
            out_pending[buf].wait()
            out_pending[buf] = None

        cbuf = in_bufs[buf]
        obuf = out_bufs[buf]
        k = _chunk_of(r, wid)

        def gather_chunk(cbuf=cbuf, obuf=obuf):
            @plsc.parallel_loop(0, CHUNK, 16, unroll=8)
            def _gather(i):
                ri = cbuf[0, pl.ds(i, 16)]
                ci = cbuf[1, pl.ds(i, 16)]
                r_ = plsc.load_gather(tab_v, [ri])
                c_ = plsc.load_gather(tab_v, [ci])
                obuf[pl.ds(i, 16)] = r_ * c_

        if r < FULL_ROUNDS:
            gather_chunk()
            out_pending[buf] = pltpu.async_copy(
                obuf, out_hbm.at[pl.ds(k * CHUNK, CHUNK)], sems_out[buf]
            )
        else:

            def tail(cbuf=cbuf, obuf=obuf, k=k):
                gather_chunk(cbuf, obuf)
                pltpu.sync_copy(obuf, out_hbm.at[pl.ds(k * CHUNK, CHUNK)])

            pl.when(wid < EXTRA)(tail)

    for cp in out_pending:
        if cp is not None:
            cp.wait()


def kernel(edge_index):
    ei = edge_index.astype(jnp.int32)
    partials = _degree_kernel(ei)
    deg_inv = pl.pallas_call(
        _reduce_rsqrt_body,
        out_shape=jax.ShapeDtypeStruct((N_ROWS, 128), jnp.float32),
    )(partials.reshape(NW, N_ROWS, 128))
    return _norm_kernel(ei, deg_inv.reshape(N_PAD))
